# Initial kernel scaffold; baseline (speedup 1.0000x reference)
import jax
import jax.numpy as jnp
from jax.experimental import pallas as pl


def kernel(x, adj, W1, b1, W2, b2, eps):
    raise NotImplementedError("write your pallas kernel here")


# speedup vs baseline: 1.4904x; 1.4904x over previous
import jax
import jax.numpy as jnp
from jax.experimental import pallas as pl
from jax.experimental.pallas import tpu as pltpu

N = 10000
NFEAT = 128
NHID = 16

BM = 400


def _xw_kernel(x_ref, w_ref, h_ref):
    h_ref[...] = jnp.dot(x_ref[...], w_ref[...],
                         preferred_element_type=jnp.float32)


def _gcn_kernel(adj_ref, h_ref, b_ref, eps_ref, z_ref, mu_ref, std_ref):
    acc = jnp.dot(adj_ref[...], h_ref[...],
                  preferred_element_type=jnp.float32)
    r = jnp.maximum(acc + b_ref[...], 0.0)
    mu = r[:, :NHID]
    std = jnp.exp(0.5 * r[:, NHID:])
    mu_ref[...] = mu
    std_ref[...] = std
    z_ref[...] = mu + std * eps_ref[...]


def kernel(x, adj, W1, b1, W2, b2, eps):
    Wcat = jnp.concatenate([W1, W2], axis=1)
    bcat = jnp.concatenate([b1, b2]).reshape(1, 2 * NHID)

    H = pl.pallas_call(
        _xw_kernel,
        grid=(N // BM,),
        in_specs=[
            pl.BlockSpec((BM, NFEAT), lambda m: (m, 0)),
            pl.BlockSpec((NFEAT, 2 * NHID), lambda m: (0, 0)),
        ],
        out_specs=pl.BlockSpec((BM, 2 * NHID), lambda m: (m, 0)),
        out_shape=jax.ShapeDtypeStruct((N, 2 * NHID), jnp.float32),
    )(x, Wcat)

    z, mu, std = pl.pallas_call(
        _gcn_kernel,
        grid=(N // BM,),
        in_specs=[
            pl.BlockSpec((BM, N), lambda m: (m, 0)),
            pl.BlockSpec((N, 2 * NHID), lambda m: (0, 0)),
            pl.BlockSpec((1, 2 * NHID), lambda m: (0, 0)),
            pl.BlockSpec((BM, NHID), lambda m: (m, 0)),
        ],
        out_specs=[
            pl.BlockSpec((BM, NHID), lambda m: (m, 0)),
            pl.BlockSpec((BM, NHID), lambda m: (m, 0)),
            pl.BlockSpec((BM, NHID), lambda m: (m, 0)),
        ],
        out_shape=[
            jax.ShapeDtypeStruct((N, NHID), jnp.float32),
            jax.ShapeDtypeStruct((N, NHID), jnp.float32),
            jax.ShapeDtypeStruct((N, NHID), jnp.float32),
        ],
        compiler_params=pltpu.CompilerParams(
            dimension_semantics=("arbitrary",),
        ),
    )(adj, H, bcat, eps)
    return (z, mu, std)
